# baseline (speedup 1.0000x reference)
import functools

import jax
import jax.numpy as jnp
from jax import lax
from jax.experimental import pallas as pl
from jax.experimental.pallas import tpu as pltpu
from jax.experimental.pallas import tpu_sc as plsc

N = 10000
NP = 10240
E = 320000
EROWS = 2560
EROWS_A = 2688
EP = EROWS_A * 128
ROWS_PER = EROWS // 32
K0 = 136
K1 = 24
R0 = 16 * K0
G = 256
D1 = 32
D2 = 64
DW = 8
NB = 4


@functools.cache
def _get_deg_kernel():
    mesh = plsc.VectorSubcoreMesh(core_axis_name="c", subcore_axis_name="s")
    return functools.partial(
        pl.kernel,
        out_type=jax.ShapeDtypeStruct((2, NP, DW), jnp.float32),
        mesh=mesh,
        scratch_types=[
            pltpu.VMEM((ROWS_PER, 128), jnp.int32),
            pltpu.VMEM((640, DW), jnp.float32),
            pltpu.VMEM_SHARED((NP, DW), jnp.float32),
        ],
        compiler_params=pltpu.CompilerParams(use_tc_tiling_on_sc=False),
    )(_deg_body)


def _deg_body(dst_hbm, degp_hbm, dst_v, ones_v, deg_sh):
    cid = lax.axis_index("c")
    sid = lax.axis_index("s")
    w = cid * 16 + sid

    def fill(val):
        def body(i, carry):
            ones_v[i, pl.ds(0, 16)] = jnp.full((16,), val, jnp.float32)
            return carry
        return body

    lax.fori_loop(0, 640, fill(0.0), 0)
    pltpu.sync_copy(ones_v, deg_sh.at[pl.ds(sid * 640, 640), :])
    plsc.subcore_barrier()

    lax.fori_loop(0, 128, fill(1.0), 0)
    pltpu.sync_copy(dst_hbm.at[pl.ds(w * ROWS_PER, ROWS_PER)], dst_v)

    def row_body(k, carry):
        pltpu.sync_copy(ones_v.at[pl.ds(0, 128), :],
                        deg_sh.at[dst_v.at[k]], add=True)
        return carry

    lax.fori_loop(0, ROWS_PER, row_body, 0)
    plsc.subcore_barrier()

    pltpu.sync_copy(deg_sh.at[pl.ds(sid * 640, 640), :], ones_v)
    pltpu.sync_copy(ones_v, degp_hbm.at[cid, pl.ds(sid * 640, 640), :])


@functools.cache
def _make_scatter(D, spmem_g):
    mesh = plsc.VectorSubcoreMesh(core_axis_name="c", subcore_axis_name="s")
    K = ROWS_PER

    scratch = [
        pltpu.VMEM((K0, 128), jnp.int32),
        pltpu.VMEM((K0, 128), jnp.int32),
        [pltpu.VMEM((128, D), jnp.float32) for _ in range(NB)],
        pltpu.VMEM_SHARED((NP, D), jnp.float32),
        [pltpu.SemaphoreType.DMA for _ in range(NB)],
        [pltpu.SemaphoreType.DMA for _ in range(NB)],
    ]
    if spmem_g:
        scratch.append(pltpu.VMEM_SHARED((NP, D), jnp.float32))

    @functools.partial(
        pl.kernel,
        out_type=jax.ShapeDtypeStruct((2, NP, D), jnp.float32),
        mesh=mesh,
        scratch_types=scratch,
        compiler_params=pltpu.CompilerParams(use_tc_tiling_on_sc=False),
    )
    def scat(src_hbm, dst_hbm, g_hbm, accp_hbm,
             src_v, dst_v, bufs, acc_sh, sems, ssems, *maybe_gsh):
        cid = lax.axis_index("c")
        sid = lax.axis_index("s")
        g_src = maybe_gsh[0] if spmem_g else g_hbm

        def zrow(i, carry):
            for j in range(D // 16):
                bufs[0][i, pl.ds(j * 16, 16)] = jnp.zeros((16,), jnp.float32)
            return carry

        with jax.named_scope("ph_zero"):
            lax.fori_loop(0, 128, zrow, 0)
            for t in range(5):
                r0 = sid * 640 + t * 128
                pltpu.sync_copy(bufs[0], acc_sh.at[pl.ds(r0, 128), :])
                if spmem_g:
                    pltpu.sync_copy(g_hbm.at[pl.ds(r0, 128), :], bufs[1])
                    pltpu.sync_copy(bufs[1], maybe_gsh[0].at[pl.ds(r0, 128), :])
            plsc.subcore_barrier()

        H = NB // 2

        def gather_start(k, b):
            pltpu.async_copy(g_src.at[src_v.at[k]], bufs[b], sems[b])

        def gather_wait(b):
            pltpu.make_async_copy(g_src.at[src_v.at[0]], bufs[b], sems[b]).wait()

        def scatter_start(k, b):
            pltpu.async_copy(bufs[b], acc_sh.at[dst_v.at[k]], ssems[b], add=True)

        def scatter_wait(b):
            pltpu.make_async_copy(bufs[b], acc_sh.at[dst_v.at[0]],
                                  ssems[b]).wait()

        def pipeline(KC, base):
            with jax.named_scope("ph_idx"):
                pltpu.sync_copy(src_hbm.at[pl.ds(base, K0)], src_v)
                pltpu.sync_copy(dst_hbm.at[pl.ds(base, K0)], dst_v)
            for b in range(H):
                gather_start(b, b)
            for b in range(NB):
                bh = (b + H) % NB
                if b >= NB - H:
                    scatter_wait(bh)
                gather_start(b + H, bh)
                gather_wait(b)
                scatter_start(b, b)

            def group(g, carry):
                for b in range(NB):
                    k = g * NB + b
                    bh = (b + H) % NB
                    scatter_wait(bh)
                    gather_start(k + H, bh)
                    gather_wait(b)
                    scatter_start(k, b)
                return carry

            lax.fori_loop(1, KC // NB - 1, group, 0)

            for b in range(NB):
                k = KC - NB + b
                if b < H:
                    bh = (b + H) % NB
                    scatter_wait(bh)
                    gather_start(k + H, bh)
                gather_wait(b)
                scatter_start(k, b)
            for b in range(NB):
                scatter_wait(b)

        base = jnp.where(cid == 0, sid * K0, R0 + sid * K1)
        kc = jnp.where(cid == 0, K0, K1)
        with jax.named_scope("ph_edges"):
            pipeline(kc, base)
        with jax.named_scope("ph_barrier2"):
            plsc.subcore_barrier()

        with jax.named_scope("ph_copyout"):
            for t in range(5):
                r0 = sid * 640 + t * 128
                pltpu.sync_copy(acc_sh.at[pl.ds(r0, 128), :], bufs[0])
                pltpu.sync_copy(bufs[0], accp_hbm.at[cid, pl.ds(r0, 128), :])

    return scat


def _tc1_body(x_ref, w1_ref, degp_ref, g1_ref, dinv_ref):
    deg = degp_ref[0, :N, 0] + degp_ref[1, :N, 0] + 1.0
    dinv = lax.rsqrt(deg).reshape(N, 1)
    h = jnp.dot(x_ref[...], w1_ref[...], preferred_element_type=jnp.float32)
    g1_ref[...] = jnp.concatenate(
        [h * dinv, jnp.zeros((N, D2 - D1), jnp.float32)], axis=1)
    dinv_ref[...] = dinv


def _tc1(x, W1, degp):
    return pl.pallas_call(
        _tc1_body,
        out_shape=[
            jax.ShapeDtypeStruct((N, D2), jnp.float32),
            jax.ShapeDtypeStruct((N, 1), jnp.float32),
        ],
    )(x, W1, degp)


def _tc2_body(g1_ref, accp_ref, dinv_ref, b1_ref, w2_ref, g2_ref):
    acc = (accp_ref[0, :N, :D1] + accp_ref[1, :N, :D1] + g1_ref[:, :D1])
    dinv = dinv_ref[...]
    o = jnp.maximum(acc * dinv + b1_ref[...][None, :], 0.0)
    h2 = jnp.dot(o, w2_ref[...], preferred_element_type=jnp.float32)
    g2_ref[...] = h2 * dinv


def _tc2(g1, accp1, dinv, b1, W2):
    return pl.pallas_call(
        _tc2_body,
        out_shape=jax.ShapeDtypeStruct((N, D2), jnp.float32),
    )(g1, accp1, dinv, b1, W2)


def _tc3_body(g2_ref, accp_ref, dinv_ref, b2_ref, bi_ref, out_ref):
    acc = accp_ref[0, :N, :] + accp_ref[1, :N, :] + g2_ref[...]
    pre = acc * dinv_ref[...] + b2_ref[...][None, :]
    sp = jnp.maximum(pre, 0.0) + jnp.log1p(jnp.exp(-jnp.abs(pre)))
    m = pre * jnp.tanh(sp)
    gid = lax.broadcasted_iota(jnp.int32, (1, G), 1)
    onehot = (bi_ref[...] == gid).astype(jnp.float32)
    sums = lax.dot_general(
        onehot, m, dimension_numbers=(((0,), (0,)), ((), ())),
        preferred_element_type=jnp.float32,
    )
    cnt = jnp.sum(onehot, axis=0)
    out_ref[...] = sums / jnp.maximum(cnt, 1.0)[:, None]


def _tc3(g2, accp2, dinv, b2, bi2d):
    return pl.pallas_call(
        _tc3_body,
        out_shape=jax.ShapeDtypeStruct((G, D2), jnp.float32),
    )(g2, accp2, dinv, b2, bi2d)


def kernel(x, edge_index, batch_index, W1, b1, W2, b2):
    src = edge_index[0]
    dst = edge_index[1]
    pad = EP - E
    srcp = jnp.concatenate(
        [src, jnp.zeros((pad,), jnp.int32)]).reshape(EROWS_A, 128)
    dstp = jnp.concatenate(
        [dst, jnp.full((pad,), N, jnp.int32)]).reshape(EROWS_A, 128)
    bip = batch_index.reshape(N, 1)

    degp = _get_deg_kernel()(dstp)
    g1, dinv = _tc1(x, W1, degp)
    accp1 = _make_scatter(D2, False)(srcp, dstp, g1)
    g2 = _tc2(g1, accp1, dinv, b1, W2)
    accp2 = _make_scatter(D2, False)(srcp, dstp, g2)
    return _tc3(g2, accp2, dinv, b2, bip)


# speedup vs baseline: 1.0751x; 1.0751x over previous
import functools

import jax
import jax.numpy as jnp
from jax import lax
from jax.experimental import pallas as pl
from jax.experimental.pallas import tpu as pltpu
from jax.experimental.pallas import tpu_sc as plsc

N = 10000
NP = 10240
E = 320000
EROWS = 2560
EROWS_A = 2688
EP = EROWS_A * 128
ROWS_PER = EROWS // 32
K0 = 160
G = 256
D1 = 32
D2 = 64
DW = 8
NB = 4


@functools.cache
def _get_deg_kernel():
    mesh = plsc.VectorSubcoreMesh(core_axis_name="c", subcore_axis_name="s")
    return functools.partial(
        pl.kernel,
        out_type=jax.ShapeDtypeStruct((2, NP, DW), jnp.float32),
        mesh=mesh,
        scratch_types=[
            pltpu.VMEM((ROWS_PER, 128), jnp.int32),
            pltpu.VMEM((640, DW), jnp.float32),
            pltpu.VMEM_SHARED((NP, DW), jnp.float32),
        ],
        compiler_params=pltpu.CompilerParams(use_tc_tiling_on_sc=False),
    )(_deg_body)


def _deg_body(dst_hbm, degp_hbm, dst_v, ones_v, deg_sh):
    cid = lax.axis_index("c")
    sid = lax.axis_index("s")
    w = cid * 16 + sid

    def fill(val):
        def body(i, carry):
            ones_v[i, pl.ds(0, 16)] = jnp.full((16,), val, jnp.float32)
            return carry
        return body

    lax.fori_loop(0, 640, fill(0.0), 0)
    pltpu.sync_copy(ones_v, deg_sh.at[pl.ds(sid * 640, 640), :])
    plsc.subcore_barrier()

    lax.fori_loop(0, 128, fill(1.0), 0)
    pltpu.sync_copy(dst_hbm.at[pl.ds(w * ROWS_PER, ROWS_PER)], dst_v)

    def row_body(k, carry):
        pltpu.sync_copy(ones_v.at[pl.ds(0, 128), :],
                        deg_sh.at[dst_v.at[k]], add=True)
        return carry

    lax.fori_loop(0, ROWS_PER, row_body, 0)
    plsc.subcore_barrier()

    pltpu.sync_copy(deg_sh.at[pl.ds(sid * 640, 640), :], ones_v)
    pltpu.sync_copy(ones_v, degp_hbm.at[cid, pl.ds(sid * 640, 640), :])


@functools.cache
def _make_scatter(D, spmem_g):
    mesh = plsc.VectorSubcoreMesh(core_axis_name="c", subcore_axis_name="s")
    K = ROWS_PER

    scratch = [
        pltpu.VMEM((K0, 128), jnp.int32),
        pltpu.VMEM((K0, 128), jnp.int32),
        [pltpu.VMEM((128, D), jnp.float32) for _ in range(NB)],
        pltpu.VMEM_SHARED((NP, D), jnp.float32),
        [pltpu.SemaphoreType.DMA for _ in range(NB)],
        [pltpu.SemaphoreType.DMA for _ in range(NB)],
    ]
    if spmem_g:
        scratch.append(pltpu.VMEM_SHARED((NP, D), jnp.float32))

    @functools.partial(
        pl.kernel,
        out_type=jax.ShapeDtypeStruct((2, NP, D), jnp.float32),
        mesh=mesh,
        scratch_types=scratch,
        compiler_params=pltpu.CompilerParams(use_tc_tiling_on_sc=False),
    )
    def scat(src_hbm, dst_hbm, g_hbm, accp_hbm,
             src_v, dst_v, bufs, acc_sh, sems, ssems, *maybe_gsh):
        cid = lax.axis_index("c")
        sid = lax.axis_index("s")
        g_src = maybe_gsh[0] if spmem_g else g_hbm

        def zrow(i, carry):
            for j in range(D // 16):
                bufs[0][i, pl.ds(j * 16, 16)] = jnp.zeros((16,), jnp.float32)
            return carry

        with jax.named_scope("ph_zero"):
            lax.fori_loop(0, 128, zrow, 0)
            for t in range(5):
                r0 = sid * 640 + t * 128
                pltpu.sync_copy(bufs[0], acc_sh.at[pl.ds(r0, 128), :])
                if spmem_g:
                    pltpu.sync_copy(g_hbm.at[pl.ds(r0, 128), :], bufs[1])
                    pltpu.sync_copy(bufs[1], maybe_gsh[0].at[pl.ds(r0, 128), :])
            plsc.subcore_barrier()

        H = NB // 2

        def gather_start(k, b):
            pltpu.async_copy(g_src.at[src_v.at[k]], bufs[b], sems[b])

        def gather_wait(b):
            pltpu.make_async_copy(g_src.at[src_v.at[0]], bufs[b], sems[b]).wait()

        def scatter_start(k, b):
            pltpu.async_copy(bufs[b], acc_sh.at[dst_v.at[k]], ssems[b], add=True)

        def scatter_wait(b):
            pltpu.make_async_copy(bufs[b], acc_sh.at[dst_v.at[0]],
                                  ssems[b]).wait()

        def pipeline(KC, base):
            with jax.named_scope("ph_idx"):
                pltpu.sync_copy(src_hbm.at[pl.ds(base, K0)], src_v)
                pltpu.sync_copy(dst_hbm.at[pl.ds(base, K0)], dst_v)
            for b in range(H):
                gather_start(b, b)
            for b in range(NB):
                bh = (b + H) % NB
                if b >= NB - H:
                    scatter_wait(bh)
                gather_start(b + H, bh)
                gather_wait(b)
                scatter_start(b, b)

            def group(g, carry):
                for b in range(NB):
                    k = g * NB + b
                    bh = (b + H) % NB
                    scatter_wait(bh)
                    gather_start(k + H, bh)
                    gather_wait(b)
                    scatter_start(k, b)
                return carry

            lax.fori_loop(1, KC // NB - 1, group, 0)

            for b in range(NB):
                k = KC - NB + b
                if b < H:
                    bh = (b + H) % NB
                    scatter_wait(bh)
                    gather_start(k + H, bh)
                gather_wait(b)
                scatter_start(k, b)
            for b in range(NB):
                scatter_wait(b)

        @pl.when(cid == 0)
        def _():
            with jax.named_scope("ph_edges"):
                pipeline(K0, sid * K0)

        with jax.named_scope("ph_barrier2"):
            plsc.subcore_barrier()

        with jax.named_scope("ph_copyout"):
            for t in range(5):
                r0 = sid * 640 + t * 128
                pltpu.sync_copy(acc_sh.at[pl.ds(r0, 128), :], bufs[0])
                pltpu.sync_copy(bufs[0], accp_hbm.at[cid, pl.ds(r0, 128), :])

    return scat


def _tc1_body(x_ref, w1_ref, degp_ref, g1_ref, dinv_ref):
    deg = degp_ref[0, :N, 0] + degp_ref[1, :N, 0] + 1.0
    dinv = lax.rsqrt(deg).reshape(N, 1)
    h = jnp.dot(x_ref[...], w1_ref[...], preferred_element_type=jnp.float32)
    g1_ref[...] = h * dinv
    dinv_ref[...] = dinv


def _tc1(x, W1, degp):
    return pl.pallas_call(
        _tc1_body,
        out_shape=[
            jax.ShapeDtypeStruct((N, D1), jnp.float32),
            jax.ShapeDtypeStruct((N, 1), jnp.float32),
        ],
    )(x, W1, degp)


def _tc2_body(g1_ref, accp_ref, dinv_ref, b1_ref, w2_ref, g2_ref):
    acc = (accp_ref[0, :N, :] + accp_ref[1, :N, :] + g1_ref[...])
    dinv = dinv_ref[...]
    o = jnp.maximum(acc * dinv + b1_ref[...][None, :], 0.0)
    h2 = jnp.dot(o, w2_ref[...], preferred_element_type=jnp.float32)
    g2_ref[...] = h2 * dinv


def _tc2(g1, accp1, dinv, b1, W2):
    return pl.pallas_call(
        _tc2_body,
        out_shape=jax.ShapeDtypeStruct((N, D2), jnp.float32),
    )(g1, accp1, dinv, b1, W2)


def _tc3_body(g2_ref, accp_ref, dinv_ref, b2_ref, bi_ref, out_ref):
    acc = accp_ref[0, :N, :] + accp_ref[1, :N, :] + g2_ref[...]
    pre = acc * dinv_ref[...] + b2_ref[...][None, :]
    sp = jnp.maximum(pre, 0.0) + jnp.log1p(jnp.exp(-jnp.abs(pre)))
    m = pre * jnp.tanh(sp)
    gid = lax.broadcasted_iota(jnp.int32, (1, G), 1)
    onehot = (bi_ref[...] == gid).astype(jnp.float32)
    sums = lax.dot_general(
        onehot, m, dimension_numbers=(((0,), (0,)), ((), ())),
        preferred_element_type=jnp.float32,
    )
    cnt = jnp.sum(onehot, axis=0)
    out_ref[...] = sums / jnp.maximum(cnt, 1.0)[:, None]


def _tc3(g2, accp2, dinv, b2, bi2d):
    return pl.pallas_call(
        _tc3_body,
        out_shape=jax.ShapeDtypeStruct((G, D2), jnp.float32),
    )(g2, accp2, dinv, b2, bi2d)


def kernel(x, edge_index, batch_index, W1, b1, W2, b2):
    src = edge_index[0]
    dst = edge_index[1]
    pad = EP - E
    srcp = jnp.concatenate(
        [src, jnp.zeros((pad,), jnp.int32)]).reshape(EROWS_A, 128)
    dstp = jnp.concatenate(
        [dst, jnp.full((pad,), N, jnp.int32)]).reshape(EROWS_A, 128)
    bip = batch_index.reshape(N, 1)

    degp = _get_deg_kernel()(dstp)
    g1, dinv = _tc1(x, W1, degp)
    accp1 = _make_scatter(D1, False)(srcp, dstp, g1)
    g2 = _tc2(g1, accp1, dinv, b1, W2)
    accp2 = _make_scatter(D2, False)(srcp, dstp, g2)
    return _tc3(g2, accp2, dinv, b2, bip)


# speedup vs baseline: 2.3490x; 2.1849x over previous
import functools

import jax
import jax.numpy as jnp
from jax import lax
from jax.experimental import pallas as pl
from jax.experimental.pallas import tpu as pltpu
from jax.experimental.pallas import tpu_sc as plsc

N = 10000
NP = 10240
E = 320000
EROWS = 2560
EROWS_A = 2688
EP = EROWS_A * 128
ROWS_PER = EROWS // 32

G = 256
D1 = 32
D2 = 64
DW = 8
NB = 4


@functools.cache
def _get_deg_kernel():
    mesh = plsc.VectorSubcoreMesh(core_axis_name="c", subcore_axis_name="s")
    return functools.partial(
        pl.kernel,
        out_type=jax.ShapeDtypeStruct((2, NP, DW), jnp.float32),
        mesh=mesh,
        scratch_types=[
            pltpu.VMEM((ROWS_PER, 128), jnp.int32),
            pltpu.VMEM((640, DW), jnp.float32),
            pltpu.VMEM_SHARED((NP, DW), jnp.float32),
        ],
        compiler_params=pltpu.CompilerParams(use_tc_tiling_on_sc=False),
    )(_deg_body)


def _deg_body(dst_hbm, degp_hbm, dst_v, ones_v, deg_sh):
    cid = lax.axis_index("c")
    sid = lax.axis_index("s")
    w = cid * 16 + sid

    def fill(val):
        def body(i, carry):
            ones_v[i, pl.ds(0, 16)] = jnp.full((16,), val, jnp.float32)
            return carry
        return body

    lax.fori_loop(0, 640, fill(0.0), 0)
    pltpu.sync_copy(ones_v, deg_sh.at[pl.ds(sid * 640, 640), :])
    plsc.subcore_barrier()

    lax.fori_loop(0, 128, fill(1.0), 0)
    pltpu.sync_copy(dst_hbm.at[pl.ds(w * ROWS_PER, ROWS_PER)], dst_v)

    def row_body(k, carry):
        pltpu.sync_copy(ones_v.at[pl.ds(0, 128), :],
                        deg_sh.at[dst_v.at[k]], add=True)
        return carry

    lax.fori_loop(0, ROWS_PER, row_body, 0)
    plsc.subcore_barrier()

    pltpu.sync_copy(deg_sh.at[pl.ds(sid * 640, 640), :], ones_v)
    pltpu.sync_copy(ones_v, degp_hbm.at[cid, pl.ds(sid * 640, 640), :])


HK = 40


@functools.cache
def _make_scatter(D):
    mesh = plsc.VectorSubcoreMesh(core_axis_name="c", subcore_axis_name="s")

    scratch = [
        pltpu.VMEM((HK, 128), jnp.int32),
        pltpu.VMEM((HK, 128), jnp.int32),
        [pltpu.VMEM((128, D), jnp.float32) for _ in range(NB)],
        pltpu.VMEM_SHARED((NP, D), jnp.float32),
        pltpu.VMEM_SHARED((NP, D), jnp.float32),
        [pltpu.SemaphoreType.DMA for _ in range(NB)],
        [pltpu.SemaphoreType.DMA for _ in range(NB)],
    ]

    @functools.partial(
        pl.kernel,
        out_type=jax.ShapeDtypeStruct((2, NP, D), jnp.float32),
        mesh=mesh,
        scratch_types=scratch,
        compiler_params=pltpu.CompilerParams(use_tc_tiling_on_sc=False),
    )
    def scat(src_hbm, dst_hbm, g_hbm, accp_hbm,
             src_v, dst_v, bufs, acc_sh, g_sh, sems, ssems):
        cid = lax.axis_index("c")
        sid = lax.axis_index("s")
        w = cid * 16 + sid
        g_src = g_sh

        def zrow(i, carry):
            for j in range(D // 16):
                bufs[0][i, pl.ds(j * 16, 16)] = jnp.zeros((16,), jnp.float32)
            return carry

        with jax.named_scope("ph_zero"):
            lax.fori_loop(0, 128, zrow, 0)
            for t in range(5):
                r0 = sid * 640 + t * 128
                pltpu.sync_copy(bufs[0], acc_sh.at[pl.ds(r0, 128), :])
                pltpu.sync_copy(g_hbm.at[pl.ds(r0, 128), :], bufs[1])
                pltpu.sync_copy(bufs[1], g_sh.at[pl.ds(r0, 128), :])
            plsc.subcore_barrier()

        H = NB // 2

        def gather_start(k, b):
            pltpu.async_copy(g_src.at[src_v.at[k]], bufs[b], sems[b])

        def gather_wait(b):
            pltpu.make_async_copy(g_src.at[src_v.at[0]], bufs[b], sems[b]).wait()

        def scatter_start(k, b):
            pltpu.async_copy(bufs[b], acc_sh.at[dst_v.at[k]], ssems[b], add=True)

        def scatter_wait(b):
            pltpu.make_async_copy(bufs[b], acc_sh.at[dst_v.at[0]],
                                  ssems[b]).wait()

        def pipeline(base):
            with jax.named_scope("ph_idx"):
                pltpu.sync_copy(src_hbm.at[pl.ds(base, HK)], src_v)
                pltpu.sync_copy(dst_hbm.at[pl.ds(base, HK)], dst_v)
            for b in range(H):
                gather_start(b, b)
            for b in range(NB):
                bh = (b + H) % NB
                if b >= NB - H:
                    scatter_wait(bh)
                gather_start(b + H, bh)
                gather_wait(b)
                scatter_start(b, b)

            def group(g, carry):
                for b in range(NB):
                    k = g * NB + b
                    bh = (b + H) % NB
                    scatter_wait(bh)
                    gather_start(k + H, bh)
                    gather_wait(b)
                    scatter_start(k, b)
                return carry

            lax.fori_loop(1, HK // NB - 1, group, 0)

            for b in range(NB):
                k = HK - NB + b
                if b < H:
                    bh = (b + H) % NB
                    scatter_wait(bh)
                    gather_start(k + H, bh)
                gather_wait(b)
                scatter_start(k, b)
            for b in range(NB):
                scatter_wait(b)

        with jax.named_scope("ph_edges"):
            for half in range(ROWS_PER // HK):
                pipeline(w * ROWS_PER + half * HK)

        with jax.named_scope("ph_barrier2"):
            plsc.subcore_barrier()

        with jax.named_scope("ph_copyout"):
            for t in range(5):
                r0 = sid * 640 + t * 128
                pltpu.sync_copy(acc_sh.at[pl.ds(r0, 128), :], bufs[0])
                pltpu.sync_copy(bufs[0], accp_hbm.at[cid, pl.ds(r0, 128), :])

    return scat


def _tc1_body(x_ref, w1_ref, degp_ref, g1_ref, dinv_ref):
    deg = degp_ref[0, :N, 0] + degp_ref[1, :N, 0] + 1.0
    dinv = lax.rsqrt(deg).reshape(N, 1)
    h = jnp.dot(x_ref[...], w1_ref[...], preferred_element_type=jnp.float32)
    g1_ref[...] = h * dinv
    dinv_ref[...] = dinv


def _tc1(x, W1, degp):
    return pl.pallas_call(
        _tc1_body,
        out_shape=[
            jax.ShapeDtypeStruct((N, D1), jnp.float32),
            jax.ShapeDtypeStruct((N, 1), jnp.float32),
        ],
    )(x, W1, degp)


def _tc2_body(g1_ref, accp_ref, dinv_ref, b1_ref, w2_ref, g2_ref):
    acc = (accp_ref[0, :N, :] + accp_ref[1, :N, :] + g1_ref[...])
    dinv = dinv_ref[...]
    o = jnp.maximum(acc * dinv + b1_ref[...][None, :], 0.0)
    h2 = jnp.dot(o, w2_ref[...], preferred_element_type=jnp.float32)
    g2_ref[...] = h2 * dinv


def _tc2(g1, accp1, dinv, b1, W2):
    return pl.pallas_call(
        _tc2_body,
        out_shape=jax.ShapeDtypeStruct((N, D2), jnp.float32),
    )(g1, accp1, dinv, b1, W2)


def _tc3_body(g2_ref, accp_ref, dinv_ref, b2_ref, bi_ref, out_ref):
    acc = accp_ref[0, :N, :] + accp_ref[1, :N, :] + g2_ref[...]
    pre = acc * dinv_ref[...] + b2_ref[...][None, :]
    sp = jnp.maximum(pre, 0.0) + jnp.log1p(jnp.exp(-jnp.abs(pre)))
    m = pre * jnp.tanh(sp)
    gid = lax.broadcasted_iota(jnp.int32, (1, G), 1)
    onehot = (bi_ref[...] == gid).astype(jnp.float32)
    sums = lax.dot_general(
        onehot, m, dimension_numbers=(((0,), (0,)), ((), ())),
        preferred_element_type=jnp.float32,
    )
    cnt = jnp.sum(onehot, axis=0)
    out_ref[...] = sums / jnp.maximum(cnt, 1.0)[:, None]


def _tc3(g2, accp2, dinv, b2, bi2d):
    return pl.pallas_call(
        _tc3_body,
        out_shape=jax.ShapeDtypeStruct((G, D2), jnp.float32),
    )(g2, accp2, dinv, b2, bi2d)


def kernel(x, edge_index, batch_index, W1, b1, W2, b2):
    src = edge_index[0]
    dst = edge_index[1]
    pad = EP - E
    srcp = jnp.concatenate(
        [src, jnp.zeros((pad,), jnp.int32)]).reshape(EROWS_A, 128)
    dstp = jnp.concatenate(
        [dst, jnp.full((pad,), N, jnp.int32)]).reshape(EROWS_A, 128)
    bip = batch_index.reshape(N, 1)

    degp = _get_deg_kernel()(dstp)
    g1, dinv = _tc1(x, W1, degp)
    accp1 = _make_scatter(D1)(srcp, dstp, g1)
    g2 = _tc2(g1, accp1, dinv, b1, W2)
    accp2 = _make_scatter(D2)(srcp, dstp, g2)
    return _tc3(g2, accp2, dinv, b2, bip)


# speedup vs baseline: 2.3788x; 1.0127x over previous
import functools

import jax
import jax.numpy as jnp
from jax import lax
from jax.experimental import pallas as pl
from jax.experimental.pallas import tpu as pltpu
from jax.experimental.pallas import tpu_sc as plsc

N = 10000
NP = 10240
E = 320000
EROWS = 2560
EROWS_A = 2688
EP = EROWS_A * 128
ROWS_PER = EROWS // 32

G = 256
D1 = 32
D2 = 64
DW = 8
NB = 4


@functools.cache
def _get_deg_kernel():
    mesh = plsc.VectorSubcoreMesh(core_axis_name="c", subcore_axis_name="s")
    return functools.partial(
        pl.kernel,
        out_type=jax.ShapeDtypeStruct((2, NP, DW), jnp.float32),
        mesh=mesh,
        scratch_types=[
            pltpu.VMEM((ROWS_PER, 128), jnp.int32),
            pltpu.VMEM((640, DW), jnp.float32),
            pltpu.VMEM_SHARED((NP, DW), jnp.float32),
        ],
        compiler_params=pltpu.CompilerParams(use_tc_tiling_on_sc=False),
    )(_deg_body)


def _deg_body(dst_hbm, degp_hbm, dst_v, ones_v, deg_sh):
    cid = lax.axis_index("c")
    sid = lax.axis_index("s")
    w = cid * 16 + sid

    def fill(val):
        def body(i, carry):
            ones_v[i, pl.ds(0, 16)] = jnp.full((16,), val, jnp.float32)
            return carry
        return body

    lax.fori_loop(0, 640, fill(0.0), 0)
    pltpu.sync_copy(ones_v, deg_sh.at[pl.ds(sid * 640, 640), :])
    plsc.subcore_barrier()

    lax.fori_loop(0, 128, fill(1.0), 0)
    pltpu.sync_copy(dst_hbm.at[pl.ds(w * ROWS_PER, ROWS_PER)], dst_v)

    def row_body(k, carry):
        pltpu.sync_copy(ones_v.at[pl.ds(0, 128), :],
                        deg_sh.at[dst_v.at[k]], add=True)
        return carry

    lax.fori_loop(0, ROWS_PER, row_body, 0)
    plsc.subcore_barrier()

    pltpu.sync_copy(deg_sh.at[pl.ds(sid * 640, 640), :], ones_v)
    pltpu.sync_copy(ones_v, degp_hbm.at[cid, pl.ds(sid * 640, 640), :])


HK = 40


@functools.cache
def _make_scatter(D):
    mesh = plsc.VectorSubcoreMesh(core_axis_name="c", subcore_axis_name="s")

    scratch = [
        pltpu.VMEM((HK, 128), jnp.int32),
        pltpu.VMEM((HK, 128), jnp.int32),
        [pltpu.VMEM((128, D), jnp.float32) for _ in range(NB)],
        pltpu.VMEM_SHARED((NP, D), jnp.float32),
        pltpu.VMEM_SHARED((NP, D), jnp.float32),
        [pltpu.SemaphoreType.DMA for _ in range(NB)],
        [pltpu.SemaphoreType.DMA for _ in range(NB)],
    ]

    @functools.partial(
        pl.kernel,
        out_type=jax.ShapeDtypeStruct((2, NP, D), jnp.float32),
        mesh=mesh,
        scratch_types=scratch,
        compiler_params=pltpu.CompilerParams(use_tc_tiling_on_sc=False),
    )
    def scat(src_hbm, dst_hbm, g_hbm, accp_hbm,
             src_v, dst_v, bufs, acc_sh, g_sh, sems, ssems):
        cid = lax.axis_index("c")
        sid = lax.axis_index("s")
        w = cid * 16 + sid
        g_src = g_sh

        def zrow(i, carry):
            for j in range(D // 16):
                bufs[0][i, pl.ds(j * 16, 16)] = jnp.zeros((16,), jnp.float32)
            return carry

        lax.fori_loop(0, 128, zrow, 0)
        for t in range(5):
            r0 = sid * 640 + t * 128
            pltpu.sync_copy(bufs[0], acc_sh.at[pl.ds(r0, 128), :])
            pltpu.sync_copy(g_hbm.at[pl.ds(r0, 128), :], g_sh.at[pl.ds(r0, 128), :])
        plsc.subcore_barrier()

        H = NB // 2

        def gather_start(k, b):
            pltpu.async_copy(g_src.at[src_v.at[k]], bufs[b], sems[b])

        def gather_wait(b):
            pltpu.make_async_copy(g_src.at[src_v.at[0]], bufs[b], sems[b]).wait()

        def scatter_start(k, b):
            pltpu.async_copy(bufs[b], acc_sh.at[dst_v.at[k]], ssems[b], add=True)

        def scatter_wait(b):
            pltpu.make_async_copy(bufs[b], acc_sh.at[dst_v.at[0]],
                                  ssems[b]).wait()

        def pipeline(base):
            pltpu.sync_copy(src_hbm.at[pl.ds(base, HK)], src_v)
            pltpu.sync_copy(dst_hbm.at[pl.ds(base, HK)], dst_v)
            for b in range(H):
                gather_start(b, b)
            for b in range(NB):
                bh = (b + H) % NB
                if b >= NB - H:
                    scatter_wait(bh)
                gather_start(b + H, bh)
                gather_wait(b)
                scatter_start(b, b)

            def group(g, carry):
                for b in range(NB):
                    k = g * NB + b
                    bh = (b + H) % NB
                    scatter_wait(bh)
                    gather_start(k + H, bh)
                    gather_wait(b)
                    scatter_start(k, b)
                return carry

            lax.fori_loop(1, HK // NB - 1, group, 0)

            for b in range(NB):
                k = HK - NB + b
                if b < H:
                    bh = (b + H) % NB
                    scatter_wait(bh)
                    gather_start(k + H, bh)
                gather_wait(b)
                scatter_start(k, b)
            for b in range(NB):
                scatter_wait(b)

        for half in range(ROWS_PER // HK):
            pipeline(w * ROWS_PER + half * HK)
        plsc.subcore_barrier()

        for t in range(5):
            r0 = sid * 640 + t * 128
            pltpu.sync_copy(acc_sh.at[pl.ds(r0, 128), :], bufs[0])
            pltpu.sync_copy(bufs[0], accp_hbm.at[cid, pl.ds(r0, 128), :])

    return scat


def _tc1_body(x_ref, w1_ref, degp_ref, g1_ref, dinv_ref):
    deg = degp_ref[0, :N, 0] + degp_ref[1, :N, 0] + 1.0
    dinv = lax.rsqrt(deg).reshape(N, 1)
    h = jnp.dot(x_ref[...], w1_ref[...], preferred_element_type=jnp.float32)
    g1_ref[...] = h * dinv
    dinv_ref[...] = dinv


def _tc1(x, W1, degp):
    return pl.pallas_call(
        _tc1_body,
        out_shape=[
            jax.ShapeDtypeStruct((N, D1), jnp.float32),
            jax.ShapeDtypeStruct((N, 1), jnp.float32),
        ],
    )(x, W1, degp)


def _tc2_body(g1_ref, accp_ref, dinv_ref, b1_ref, w2_ref, g2_ref):
    acc = (accp_ref[0, :N, :] + accp_ref[1, :N, :] + g1_ref[...])
    dinv = dinv_ref[...]
    o = jnp.maximum(acc * dinv + b1_ref[...][None, :], 0.0)
    h2 = jnp.dot(o, w2_ref[...], preferred_element_type=jnp.float32)
    g2_ref[...] = h2 * dinv


def _tc2(g1, accp1, dinv, b1, W2):
    return pl.pallas_call(
        _tc2_body,
        out_shape=jax.ShapeDtypeStruct((N, D2), jnp.float32),
    )(g1, accp1, dinv, b1, W2)


def _tc3_body(g2_ref, accp_ref, dinv_ref, b2_ref, bi_ref, out_ref):
    acc = accp_ref[0, :N, :] + accp_ref[1, :N, :] + g2_ref[...]
    pre = acc * dinv_ref[...] + b2_ref[...][None, :]
    sp = jnp.maximum(pre, 0.0) + jnp.log1p(jnp.exp(-jnp.abs(pre)))
    m = pre * jnp.tanh(sp)
    gid = lax.broadcasted_iota(jnp.int32, (1, G), 1)
    onehot = (bi_ref[...] == gid).astype(jnp.float32)
    sums = lax.dot_general(
        onehot, m, dimension_numbers=(((0,), (0,)), ((), ())),
        preferred_element_type=jnp.float32,
    )
    cnt = jnp.sum(onehot, axis=0)
    out_ref[...] = sums / jnp.maximum(cnt, 1.0)[:, None]


def _tc3(g2, accp2, dinv, b2, bi2d):
    return pl.pallas_call(
        _tc3_body,
        out_shape=jax.ShapeDtypeStruct((G, D2), jnp.float32),
    )(g2, accp2, dinv, b2, bi2d)


def kernel(x, edge_index, batch_index, W1, b1, W2, b2):
    src = edge_index[0]
    dst = edge_index[1]
    pad = EP - E
    srcp = jnp.concatenate(
        [src, jnp.zeros((pad,), jnp.int32)]).reshape(EROWS_A, 128)
    dstp = jnp.concatenate(
        [dst, jnp.full((pad,), N, jnp.int32)]).reshape(EROWS_A, 128)
    bip = batch_index.reshape(N, 1)

    degp = _get_deg_kernel()(dstp)
    g1, dinv = _tc1(x, W1, degp)
    accp1 = _make_scatter(D1)(srcp, dstp, g1)
    g2 = _tc2(g1, accp1, dinv, b1, W2)
    accp2 = _make_scatter(D2)(srcp, dstp, g2)
    return _tc3(g2, accp2, dinv, b2, bip)
